# Initial kernel scaffold; baseline (speedup 1.0000x reference)
import jax
import jax.numpy as jnp
from jax.experimental import pallas as pl


def kernel(x, edge_index, params):
    raise NotImplementedError("write your pallas kernel here")


# speedup vs baseline: 3.1789x; 3.1789x over previous
import functools

import jax
import jax.numpy as jnp
from jax import lax
from jax.experimental import pallas as pl
from jax.experimental.pallas import tpu as pltpu
from jax.experimental.pallas import tpu_sc as plsc

N = 10000
E = 160000
C = 256
HALF = 128
NSUB = 16
NCORE = 2
CHUNK = 128
EP = 10112
NPAD = 10240
ROWS_PER_SUB = NPAD // NSUB
NCHUNK = EP // CHUNK


def _sc_agg_body(tab_hbm, src_hbm, dst_hbm, out_hbm, src_v, dst_v, rows_v,
                 acc, sem):
    c = lax.axis_index("c")
    s = lax.axis_index("s")

    pltpu.sync_copy(tab_hbm.at[pl.ds(c * NPAD + s * ROWS_PER_SUB, ROWS_PER_SUB)],
                    acc.at[pl.ds(s * ROWS_PER_SUB, ROWS_PER_SUB)])
    plsc.subcore_barrier()

    def chunk_body(j, carry):
        base = j * CHUNK
        pltpu.sync_copy(src_hbm.at[c, s, pl.ds(base, CHUNK)], src_v)
        pltpu.sync_copy(dst_hbm.at[s, pl.ds(base, CHUNK)], dst_v)
        pltpu.async_copy(tab_hbm.at[src_v], rows_v, sem).wait()
        pltpu.sync_copy(rows_v, acc.at[dst_v], add=True)
        return carry

    lax.fori_loop(0, NCHUNK, chunk_body, 0)
    plsc.subcore_barrier()

    pltpu.sync_copy(acc.at[pl.ds(s * ROWS_PER_SUB, ROWS_PER_SUB)],
                    out_hbm.at[c, pl.ds(s * ROWS_PER_SUB, ROWS_PER_SUB)])


_sc_agg = functools.partial(
    pl.kernel,
    out_type=jax.ShapeDtypeStruct((NCORE, NPAD, HALF), jnp.float32),
    mesh=plsc.VectorSubcoreMesh(core_axis_name="c", subcore_axis_name="s"),
    scratch_types=[
        pltpu.VMEM((CHUNK,), jnp.int32),
        pltpu.VMEM((CHUNK,), jnp.int32),
        pltpu.VMEM((CHUNK, HALF), jnp.float32),
        pltpu.VMEM_SHARED((NPAD, HALF), jnp.float32),
        pltpu.SemaphoreType.DMA,
    ],
)(_sc_agg_body)


def _bn_cols(y, g, b):
    m = jnp.mean(y, axis=0, keepdims=True)
    v = jnp.mean((y - m) * (y - m), axis=0, keepdims=True)
    return (y - m) * lax.rsqrt(v + 1e-5) * g + b


def _mlp_from_split(z_ref, w1_ref, b1_ref, w2_ref, b2_ref, g1_ref, be1_ref,
                    g2_ref, be2_ref):
    z = jnp.concatenate([z_ref[0, :N, :], z_ref[1, :N, :]], axis=1)
    t = jnp.maximum(jnp.dot(z, w1_ref[...],
                            preferred_element_type=jnp.float32) + b1_ref[...], 0.0)
    y = jnp.dot(t, w2_ref[...], preferred_element_type=jnp.float32) + b2_ref[...]
    y = _bn_cols(y, g1_ref[...], be1_ref[...])
    y = _bn_cols(y, g2_ref[...], be2_ref[...])
    return jnp.maximum(y, 0.0)


def _tc_layer_body(z_ref, w1_ref, b1_ref, w2_ref, b2_ref, g1_ref, be1_ref,
                   g2_ref, be2_ref, out_ref):
    h = _mlp_from_split(z_ref, w1_ref, b1_ref, w2_ref, b2_ref, g1_ref, be1_ref,
                        g2_ref, be2_ref)
    out_ref[0, :N, :] = h[:, :HALF]
    out_ref[1, :N, :] = h[:, HALF:]
    out_ref[0, N:, :] = jnp.zeros((NPAD - N, HALF), jnp.float32)
    out_ref[1, N:, :] = jnp.zeros((NPAD - N, HALF), jnp.float32)


def _tc_final_body(z_ref, w1_ref, b1_ref, w2_ref, b2_ref, g1_ref, be1_ref,
                   g2_ref, be2_ref, lw_ref, lb_ref, out_ref):
    h = _mlp_from_split(z_ref, w1_ref, b1_ref, w2_ref, b2_ref, g1_ref, be1_ref,
                        g2_ref, be2_ref)
    logits = jnp.dot(h, lw_ref[...], preferred_element_type=jnp.float32) + lb_ref[...]
    m = jnp.max(logits, axis=1, keepdims=True)
    sh = logits - m
    lse = jnp.log(jnp.sum(jnp.exp(sh), axis=1, keepdims=True))
    out_ref[...] = sh - lse


_tc_layer = pl.pallas_call(
    _tc_layer_body,
    out_shape=jax.ShapeDtypeStruct((NCORE, NPAD, HALF), jnp.float32),
)

_tc_final = pl.pallas_call(
    _tc_final_body,
    out_shape=jax.ShapeDtypeStruct((N, C), jnp.float32),
)


def _layer_args(p):
    return (p["W1"], p["b1"].reshape(1, C), p["W2"], p["b2"].reshape(1, C),
            p["bn_mlp_g"].reshape(1, C), p["bn_mlp_b"].reshape(1, C),
            p["bn_out_g"].reshape(1, C), p["bn_out_b"].reshape(1, C))


def kernel(x, edge_index, params):
    src = edge_index[0]
    dst = edge_index[1]
    pad = NSUB * EP - E
    srcp = jnp.concatenate([src, jnp.full((pad,), N, jnp.int32)])
    dstp = jnp.concatenate([dst, jnp.zeros((pad,), jnp.int32)])
    src2 = jnp.stack([srcp, srcp + NPAD]).reshape(NCORE, NSUB, EP)
    dst2 = dstp.reshape(NSUB, EP)

    tab = jnp.zeros((NCORE, NPAD, HALF), jnp.float32)
    tab = tab.at[:, :N, :].set(jnp.stack([x[:, :HALF], x[:, HALF:]]))

    z = _sc_agg(tab.reshape(NCORE * NPAD, HALF), src2, dst2)
    h = _tc_layer(z, *_layer_args(params["conv0"]))
    z = _sc_agg(h.reshape(NCORE * NPAD, HALF), src2, dst2)
    h = _tc_layer(z, *_layer_args(params["conv1"]))
    z = _sc_agg(h.reshape(NCORE * NPAD, HALF), src2, dst2)
    return _tc_final(z, *_layer_args(params["conv2"]),
                     params["lin_W"], params["lin_b"].reshape(1, C))


# speedup vs baseline: 3.3821x; 1.0639x over previous
import functools

import jax
import jax.numpy as jnp
from jax import lax
from jax.experimental import pallas as pl
from jax.experimental.pallas import tpu as pltpu
from jax.experimental.pallas import tpu_sc as plsc

N = 10000
E = 160000
C = 256
HALF = 128
NSUB = 16
NCORE = 2
CHUNK = 104
EP = 10192
NPAD = 10240
ROWS_PER_SUB = NPAD // NSUB
NCHUNK = EP // CHUNK
NGROUP = NCHUNK // 2


def _sc_agg_body(tab_hbm, src_hbm, dst_hbm, out_hbm, src_v, dst_v,
                 rows0, rows1, acc, gs0, gs1, ss0, ss1):
    c = lax.axis_index("c")
    s = lax.axis_index("s")
    rows = [rows0, rows1]
    gsem = [gs0, gs1]
    ssem = [ss0, ss1]

    pltpu.sync_copy(tab_hbm.at[pl.ds(c * NPAD + s * ROWS_PER_SUB, ROWS_PER_SUB)],
                    acc.at[pl.ds(s * ROWS_PER_SUB, ROWS_PER_SUB)])
    pltpu.sync_copy(src_hbm.at[c, s], src_v)
    pltpu.sync_copy(dst_hbm.at[s], dst_v)
    plsc.subcore_barrier()

    def g_start(j, b):
        pltpu.async_copy(tab_hbm.at[src_v.at[pl.ds(j * CHUNK, CHUNK)]], rows[b], gsem[b])

    def g_wait(j, b):
        pltpu.make_async_copy(tab_hbm.at[src_v.at[pl.ds(j * CHUNK, CHUNK)]], rows[b], gsem[b]).wait()

    def s_start(j, b):
        pltpu.async_copy(rows[b], acc.at[dst_v.at[pl.ds(j * CHUNK, CHUNK)]], ssem[b], add=True)

    def s_wait(j, b):
        pltpu.make_async_copy(rows[b], acc.at[dst_v.at[pl.ds(j * CHUNK, CHUNK)]], ssem[b]).wait()

    g_start(0, 0)

    g_wait(0, 0)
    s_start(0, 0)
    g_start(1, 1)
    g_wait(1, 1)
    s_start(1, 1)
    s_wait(0, 0)
    g_start(2, 0)

    def pair_body(t, carry):
        j = 2 * t
        g_wait(j, 0)
        s_start(j, 0)
        s_wait(j - 1, 1)
        g_start(j + 1, 1)
        g_wait(j + 1, 1)
        s_start(j + 1, 1)
        s_wait(j, 0)
        g_start(j + 2, 0)
        return carry

    lax.fori_loop(1, NGROUP - 1, pair_body, 0)

    j = NCHUNK - 2
    g_wait(j, 0)
    s_start(j, 0)
    s_wait(j - 1, 1)
    g_start(j + 1, 1)
    g_wait(j + 1, 1)
    s_start(j + 1, 1)
    s_wait(j, 0)
    s_wait(j + 1, 1)

    plsc.subcore_barrier()
    pltpu.sync_copy(acc.at[pl.ds(s * ROWS_PER_SUB, ROWS_PER_SUB)],
                    out_hbm.at[c, pl.ds(s * ROWS_PER_SUB, ROWS_PER_SUB)])


_sc_agg = functools.partial(
    pl.kernel,
    out_type=jax.ShapeDtypeStruct((NCORE, NPAD, HALF), jnp.float32),
    mesh=plsc.VectorSubcoreMesh(core_axis_name="c", subcore_axis_name="s"),
    scratch_types=[
        pltpu.VMEM((EP,), jnp.int32),
        pltpu.VMEM((EP,), jnp.int32),
        pltpu.VMEM((CHUNK, HALF), jnp.float32),
        pltpu.VMEM((CHUNK, HALF), jnp.float32),
        pltpu.VMEM_SHARED((NPAD, HALF), jnp.float32),
        pltpu.SemaphoreType.DMA,
        pltpu.SemaphoreType.DMA,
        pltpu.SemaphoreType.DMA,
        pltpu.SemaphoreType.DMA,
    ],
)(_sc_agg_body)


def _bn_cols(y, g, b):
    m = jnp.mean(y, axis=0, keepdims=True)
    v = jnp.mean((y - m) * (y - m), axis=0, keepdims=True)
    return (y - m) * lax.rsqrt(v + 1e-5) * g + b


def _mlp_from_split(z_ref, w1_ref, b1_ref, w2_ref, b2_ref, g1_ref, be1_ref,
                    g2_ref, be2_ref):
    z = jnp.concatenate([z_ref[0, :N, :], z_ref[1, :N, :]], axis=1)
    t = jnp.maximum(jnp.dot(z, w1_ref[...],
                            preferred_element_type=jnp.float32) + b1_ref[...], 0.0)
    y = jnp.dot(t, w2_ref[...], preferred_element_type=jnp.float32) + b2_ref[...]
    y = _bn_cols(y, g1_ref[...], be1_ref[...])
    y = _bn_cols(y, g2_ref[...], be2_ref[...])
    return jnp.maximum(y, 0.0)


def _tc_layer_body(z_ref, w1_ref, b1_ref, w2_ref, b2_ref, g1_ref, be1_ref,
                   g2_ref, be2_ref, out_ref):
    h = _mlp_from_split(z_ref, w1_ref, b1_ref, w2_ref, b2_ref, g1_ref, be1_ref,
                        g2_ref, be2_ref)
    out_ref[0, :N, :] = h[:, :HALF]
    out_ref[1, :N, :] = h[:, HALF:]
    out_ref[0, N:, :] = jnp.zeros((NPAD - N, HALF), jnp.float32)
    out_ref[1, N:, :] = jnp.zeros((NPAD - N, HALF), jnp.float32)


def _tc_final_body(z_ref, w1_ref, b1_ref, w2_ref, b2_ref, g1_ref, be1_ref,
                   g2_ref, be2_ref, lw_ref, lb_ref, out_ref):
    h = _mlp_from_split(z_ref, w1_ref, b1_ref, w2_ref, b2_ref, g1_ref, be1_ref,
                        g2_ref, be2_ref)
    logits = jnp.dot(h, lw_ref[...], preferred_element_type=jnp.float32) + lb_ref[...]
    m = jnp.max(logits, axis=1, keepdims=True)
    sh = logits - m
    lse = jnp.log(jnp.sum(jnp.exp(sh), axis=1, keepdims=True))
    out_ref[...] = sh - lse


_tc_layer = pl.pallas_call(
    _tc_layer_body,
    out_shape=jax.ShapeDtypeStruct((NCORE, NPAD, HALF), jnp.float32),
)

_tc_final = pl.pallas_call(
    _tc_final_body,
    out_shape=jax.ShapeDtypeStruct((N, C), jnp.float32),
)


def _layer_args(p):
    return (p["W1"], p["b1"].reshape(1, C), p["W2"], p["b2"].reshape(1, C),
            p["bn_mlp_g"].reshape(1, C), p["bn_mlp_b"].reshape(1, C),
            p["bn_out_g"].reshape(1, C), p["bn_out_b"].reshape(1, C))


def kernel(x, edge_index, params):
    src = edge_index[0]
    dst = edge_index[1]
    pad = NSUB * EP - E
    srcp = jnp.concatenate([src, jnp.full((pad,), N, jnp.int32)])
    dstp = jnp.concatenate([dst, jnp.zeros((pad,), jnp.int32)])
    src2 = jnp.stack([srcp, srcp + NPAD]).reshape(NCORE, NSUB, EP)
    dst2 = dstp.reshape(NSUB, EP)

    tab = jnp.zeros((NCORE, NPAD, HALF), jnp.float32)
    tab = tab.at[:, :N, :].set(jnp.stack([x[:, :HALF], x[:, HALF:]]))

    z = _sc_agg(tab.reshape(NCORE * NPAD, HALF), src2, dst2)
    h = _tc_layer(z, *_layer_args(params["conv0"]))
    z = _sc_agg(h.reshape(NCORE * NPAD, HALF), src2, dst2)
    h = _tc_layer(z, *_layer_args(params["conv1"]))
    z = _sc_agg(h.reshape(NCORE * NPAD, HALF), src2, dst2)
    return _tc_final(z, *_layer_args(params["conv2"]),
                     params["lin_W"], params["lin_b"].reshape(1, C))
